# baseline (speedup 1.0000x reference)
import jax
import jax.numpy as jnp
from jax.experimental import pallas as pl
from jax.experimental.pallas import tpu as pltpu

CHAR = 256
B = 4096
S = 50
BB = 128
NSTEP = B // BB


def _onehot_block(xt_ref, o_ref):
    i = pl.program_id(0)
    xt = xt_ref[...]
    lane = jax.lax.broadcasted_iota(jnp.int32, (S, BB, CHAR), 2)
    oh = xt[:, :, None] == lane
    bcol = jax.lax.broadcasted_iota(jnp.int32, (1, BB, 1), 1) + i * BB
    oh = jnp.logical_and(oh, bcol != 0)
    o_ref[...] = oh.astype(jnp.int32)


def kernel(x):
    xt = x.T
    out_t = pl.pallas_call(
        _onehot_block,
        grid=(NSTEP,),
        in_specs=[pl.BlockSpec((S, BB), lambda i: (0, i))],
        out_specs=pl.BlockSpec((S, BB, CHAR), lambda i: (0, i, 0)),
        out_shape=jax.ShapeDtypeStruct((S, B, CHAR), jnp.int32),
    )(xt)
    return jnp.swapaxes(out_t, 0, 1)
